# baseline (speedup 1.0000x reference)
import jax
import jax.numpy as jnp
from jax import lax
from jax.experimental import pallas as pl
from jax.experimental.pallas import tpu as pltpu
from jax.experimental.pallas import tpu_sc as plsc

NC = 2
NS = 16
NW = NC * NS
CH = 128
BN = 1024


def _deg_body(col3, cnt_out, colbuf, ones_v, zb, deg_sh):
    cid = lax.axis_index("c")
    sid = lax.axis_index("s")
    wid = sid * NC + cid
    npad = deg_sh.shape[0]
    pt = npad // NS

    for i in range(ones_v.shape[0] // 16):
        ones_v[pl.ds(i * 16, 16)] = jnp.full((16,), 1.0, jnp.float32)

    def zloop(i, c):
        zb[pl.ds(i * 16, 16)] = jnp.zeros((16,), jnp.float32)
        return c

    lax.fori_loop(0, pt // 16, zloop, 0)
    pltpu.sync_copy(zb, deg_sh.at[pl.ds(sid * pt, pt)])
    plsc.subcore_barrier()

    def sloop(s, c):
        pltpu.sync_copy(col3.at[wid, s], colbuf)

        def eloop(k, c2):
            pltpu.sync_copy(ones_v.at[pl.ds(0, colbuf.shape[1])],
                            deg_sh.at[colbuf.at[k]], add=True)
            return c2

        lax.fori_loop(0, colbuf.shape[0], eloop, 0)
        return c

    lax.fori_loop(0, col3.shape[1], sloop, 0)
    plsc.subcore_barrier()
    pltpu.sync_copy(deg_sh.at[pl.ds(sid * pt, pt)],
                    cnt_out.at[cid, pl.ds(sid * pt, pt)])


def _deg_call(col3, npad):
    return pl.kernel(
        _deg_body,
        out_type=jax.ShapeDtypeStruct((NC, npad), jnp.float32),
        mesh=plsc.VectorSubcoreMesh(core_axis_name="c", subcore_axis_name="s",
                                    num_cores=NC, num_subcores=NS),
        scratch_types=[
            pltpu.VMEM(col3.shape[2:], jnp.int32),
            pltpu.VMEM((128,), jnp.float32),
            pltpu.VMEM((npad // NS,), jnp.float32),
            pltpu.VMEM_SHARED((npad,), jnp.float32),
        ],
    )(col3)


NBUF = 2


def _scat_body(g, row3, col3, out, rowbuf, colbuf,
               b0, b1, g0, g1, s0, s1, acc_sh):
    bufs = (b0, b1)
    gsem = (g0, g1)
    ssem = (s0, s1)
    cid = lax.axis_index("c")
    sid = lax.axis_index("s")
    wid = sid * NC + cid
    npad = acc_sh.shape[0]
    pt = npad // NS
    zrows = b0.shape[0]
    nsec = row3.shape[1]

    def zloop(i, c):
        for j in range(b0.shape[1] // 16):
            b0[i, pl.ds(j * 16, 16)] = jnp.zeros((16,), jnp.float32)
        return c

    lax.fori_loop(0, zrows, zloop, 0)
    zstep = 40
    for k in range(pt // zstep):
        pltpu.sync_copy(b0.at[pl.ds(0, zstep)],
                        acc_sh.at[pl.ds(sid * pt + k * zstep, zstep)])
    plsc.subcore_barrier()

    ns = rowbuf.shape[0]
    np4 = ns // NBUF

    def sloop(s, c):
        pltpu.sync_copy(row3.at[wid, s], rowbuf)
        pltpu.sync_copy(col3.at[wid, s], colbuf)
        for b in range(NBUF):
            pltpu.async_copy(g.at[rowbuf.at[b]], bufs[b], gsem[b])

        def qloop(j, c2):
            for b in range(NBUF):
                cc = NBUF * j + b
                pltpu.make_async_copy(g.at[rowbuf.at[cc]], bufs[b],
                                      gsem[b]).wait()
                pltpu.async_copy(bufs[b], acc_sh.at[colbuf.at[cc]],
                                 ssem[b], add=True)

                @pl.when(j < np4 - 1)
                def _():
                    pltpu.make_async_copy(bufs[b], acc_sh.at[colbuf.at[cc]],
                                          ssem[b]).wait()
                    pltpu.async_copy(g.at[rowbuf.at[cc + NBUF]], bufs[b],
                                     gsem[b])
            return c2

        lax.fori_loop(0, np4, qloop, 0)
        for b in range(NBUF):
            pltpu.make_async_copy(bufs[b], acc_sh.at[colbuf.at[ns - NBUF + b]],
                                  ssem[b]).wait()
        return c

    lax.fori_loop(0, nsec, sloop, 0)
    plsc.subcore_barrier()
    pltpu.sync_copy(acc_sh.at[pl.ds(sid * pt, pt)],
                    out.at[cid, pl.ds(sid * pt, pt)])


def _scat_call(g, row3, col3):
    npad, d = g.shape
    ns = row3.shape[2]
    ch = row3.shape[3]
    return pl.kernel(
        _scat_body,
        out_type=jax.ShapeDtypeStruct((NC, npad, d), jnp.float32),
        mesh=plsc.VectorSubcoreMesh(core_axis_name="c", subcore_axis_name="s",
                                    num_cores=NC, num_subcores=NS),
        scratch_types=[
            pltpu.VMEM((ns, ch), jnp.int32),
            pltpu.VMEM((ns, ch), jnp.int32),
        ] + [pltpu.VMEM((ch, d), jnp.float32)] * NBUF
          + [pltpu.SemaphoreType.DMA] * (2 * NBUF)
          + [pltpu.VMEM_SHARED((npad, d), jnp.float32)],
    )(g, row3, col3)


def _scale1_body(x_ref, w_ref, cnt_ref, g_ref):
    deg = cnt_ref[0] + cnt_ref[1] + 1.0
    dis = lax.rsqrt(deg)
    h = jnp.dot(x_ref[...], w_ref[...], preferred_element_type=jnp.float32)
    g_ref[...] = h * dis[:, None]


def _dense2_body(s_ref, g1_ref, cnt_ref, b_ref, w_ref, g2_ref):
    deg = cnt_ref[0] + cnt_ref[1] + 1.0
    dis = lax.rsqrt(deg)[:, None]
    t = (s_ref[0] + s_ref[1] + g1_ref[...]) * dis + b_ref[...]
    z = jnp.maximum(t, 0.0)
    g2_ref[...] = jnp.dot(z, w_ref[...],
                          preferred_element_type=jnp.float32) * dis


def _final_body(s_ref, g2_ref, cnt_ref, b_ref, out_ref):
    deg = cnt_ref[0] + cnt_ref[1] + 1.0
    dis = lax.rsqrt(deg)[:, None]
    out_ref[...] = (s_ref[0] + s_ref[1] + g2_ref[...]) * dis + b_ref[...]


def _scale1(x, W, cnt):
    npad, d = x.shape
    return pl.pallas_call(
        _scale1_body,
        grid=(npad // BN,),
        in_specs=[
            pl.BlockSpec((BN, d), lambda i: (i, 0)),
            pl.BlockSpec((d, d), lambda i: (0, 0)),
            pl.BlockSpec((NC, BN), lambda i: (0, i)),
        ],
        out_specs=pl.BlockSpec((BN, d), lambda i: (i, 0)),
        out_shape=jax.ShapeDtypeStruct((npad, d), jnp.float32),
    )(x, W, cnt)


def _dense2(s, g1, cnt, b, W):
    npad, d = g1.shape
    return pl.pallas_call(
        _dense2_body,
        grid=(npad // BN,),
        in_specs=[
            pl.BlockSpec((NC, BN, d), lambda i: (0, i, 0)),
            pl.BlockSpec((BN, d), lambda i: (i, 0)),
            pl.BlockSpec((NC, BN), lambda i: (0, i)),
            pl.BlockSpec((1, d), lambda i: (0, 0)),
            pl.BlockSpec((d, d), lambda i: (0, 0)),
        ],
        out_specs=pl.BlockSpec((BN, d), lambda i: (i, 0)),
        out_shape=jax.ShapeDtypeStruct((npad, d), jnp.float32),
    )(s, g1, cnt, b, W)


def _final(s, g2, cnt, b):
    npad, d = g2.shape
    return pl.pallas_call(
        _final_body,
        grid=(npad // BN,),
        in_specs=[
            pl.BlockSpec((NC, BN, d), lambda i: (0, i, 0)),
            pl.BlockSpec((BN, d), lambda i: (i, 0)),
            pl.BlockSpec((NC, BN), lambda i: (0, i)),
            pl.BlockSpec((1, d), lambda i: (0, 0)),
        ],
        out_specs=pl.BlockSpec((BN, d), lambda i: (i, 0)),
        out_shape=jax.ShapeDtypeStruct((npad, d), jnp.float32),
    )(s, g2, cnt, b)


def kernel(x, edge_index, W1, b1, W2, b2):
    n, d = x.shape
    e = edge_index.shape[1]
    epw = e // NW
    nsec = 5
    npad = -(-n // BN) * BN
    epp = -(-epw // (nsec * CH)) * (nsec * CH)
    ns = epp // CH // nsec
    row_w = edge_index[0].reshape(NW, epw)
    col_w = edge_index[1].reshape(NW, epw)
    row_w = jnp.pad(row_w, ((0, 0), (0, epp - epw)))
    col_w = jnp.pad(col_w, ((0, 0), (0, epp - epw)),
                    constant_values=npad - 1)
    row3 = row_w.reshape(NW, nsec, ns, CH)
    col3 = col_w.reshape(NW, nsec, ns, CH)
    xp = jnp.pad(x, ((0, npad - n), (0, 0)))

    cnt = _deg_call(col3, npad)
    g1 = _scale1(xp, W1, cnt)
    s1 = _scat_call(g1, row3, col3)
    g2 = _dense2(s1, g1, cnt, b1.reshape(1, d), W2)
    s2 = _scat_call(g2, row3, col3)
    return _final(s2, g2, cnt, b2.reshape(1, d))[:n]


# speedup vs baseline: 2.7786x; 2.7786x over previous
import jax
import jax.numpy as jnp
from jax import lax
from jax.experimental import pallas as pl
from jax.experimental.pallas import tpu as pltpu
from jax.experimental.pallas import tpu_sc as plsc

NC = 2
NS = 16
NW = NC * NS
CH = 100
BN = 1024


def _deg_body(col3, cnt_out, colbuf, ones_v, zb, d0, d1, deg_sh):
    dsem = (d0, d1)
    cid = lax.axis_index("c")
    sid = lax.axis_index("s")
    wid = sid * NC + cid
    npad = deg_sh.shape[0]
    pt = npad // NS

    for i in range(ones_v.shape[0] // 16):
        ones_v[pl.ds(i * 16, 16)] = jnp.full((16,), 1.0, jnp.float32)

    def zloop(i, c):
        zb[pl.ds(i * 16, 16)] = jnp.zeros((16,), jnp.float32)
        return c

    lax.fori_loop(0, pt // 16, zloop, 0)
    pltpu.sync_copy(zb, deg_sh.at[pl.ds(sid * pt, pt)])
    plsc.subcore_barrier()

    ones = ones_v.at[pl.ds(0, colbuf.shape[1])]

    def sloop(s, c):
        pltpu.sync_copy(col3.at[wid, s], colbuf)

        def eloop(k, c2):
            for b in range(2):
                cc = 2 * k + b

                @pl.when(cc >= 2)
                def _():
                    pltpu.make_async_copy(ones, deg_sh.at[colbuf.at[cc]],
                                          dsem[b]).wait()

                pltpu.async_copy(ones, deg_sh.at[colbuf.at[cc]], dsem[b],
                                 add=True)
            return c2

        lax.fori_loop(0, colbuf.shape[0] // 2, eloop, 0)
        for b in range(2):
            nc2 = colbuf.shape[0] - 2 + b
            pltpu.make_async_copy(ones, deg_sh.at[colbuf.at[nc2]],
                                  dsem[b]).wait()
        return c

    lax.fori_loop(0, col3.shape[1], sloop, 0)
    plsc.subcore_barrier()
    pltpu.sync_copy(deg_sh.at[pl.ds(sid * pt, pt)],
                    cnt_out.at[cid, pl.ds(sid * pt, pt)])


def _deg_call(col3, npad):
    return pl.kernel(
        _deg_body,
        out_type=jax.ShapeDtypeStruct((NC, npad), jnp.float32),
        mesh=plsc.VectorSubcoreMesh(core_axis_name="c", subcore_axis_name="s",
                                    num_cores=NC, num_subcores=NS),
        scratch_types=[
            pltpu.VMEM(col3.shape[2:], jnp.int32),
            pltpu.VMEM((128,), jnp.float32),
            pltpu.VMEM((npad // NS,), jnp.float32),
            pltpu.SemaphoreType.DMA,
            pltpu.SemaphoreType.DMA,
            pltpu.VMEM_SHARED((npad,), jnp.float32),
        ],
    )(col3)


NBUF = 2


def _scat_body(g, row3, col3, out, rowbuf, colbuf,
               b0, b1, g0, g1, s0, s1, acc_sh):
    bufs = (b0, b1)
    gsem = (g0, g1)
    ssem = (s0, s1)
    cid = lax.axis_index("c")
    sid = lax.axis_index("s")
    wid = sid * NC + cid
    npad = acc_sh.shape[0]
    pt = npad // NS
    zrows = b0.shape[0]
    nsec = row3.shape[1]

    def zloop(i, c):
        for j in range(b0.shape[1] // 16):
            b0[i, pl.ds(j * 16, 16)] = jnp.zeros((16,), jnp.float32)
        return c

    lax.fori_loop(0, zrows, zloop, 0)
    zstep = 80
    for k in range(pt // zstep):
        pltpu.sync_copy(b0.at[pl.ds(0, zstep)],
                        acc_sh.at[pl.ds(sid * pt + k * zstep, zstep)])
    plsc.subcore_barrier()

    ns = rowbuf.shape[0]
    np4 = ns // NBUF

    def sloop(s, c):
        pltpu.sync_copy(row3.at[wid, s], rowbuf)
        pltpu.sync_copy(col3.at[wid, s], colbuf)
        for b in range(NBUF):
            pltpu.async_copy(g.at[rowbuf.at[b]], bufs[b], gsem[b])

        def qloop(j, c2):
            for b in range(NBUF):
                cc = NBUF * j + b
                pltpu.make_async_copy(g.at[rowbuf.at[cc]], bufs[b],
                                      gsem[b]).wait()
                pltpu.async_copy(bufs[b], acc_sh.at[colbuf.at[cc]],
                                 ssem[b], add=True)

                @pl.when(j < np4 - 1)
                def _():
                    pltpu.make_async_copy(bufs[b], acc_sh.at[colbuf.at[cc]],
                                          ssem[b]).wait()
                    pltpu.async_copy(g.at[rowbuf.at[cc + NBUF]], bufs[b],
                                     gsem[b])
            return c2

        lax.fori_loop(0, np4, qloop, 0)
        for b in range(NBUF):
            pltpu.make_async_copy(bufs[b], acc_sh.at[colbuf.at[ns - NBUF + b]],
                                  ssem[b]).wait()
        return c

    lax.fori_loop(0, nsec, sloop, 0)
    plsc.subcore_barrier()
    pltpu.sync_copy(acc_sh.at[pl.ds(sid * pt, pt)],
                    out.at[cid, pl.ds(sid * pt, pt)])


def _scat_call(g, row3, col3):
    npad, d = g.shape
    ns = row3.shape[2]
    ch = row3.shape[3]
    return pl.kernel(
        _scat_body,
        out_type=jax.ShapeDtypeStruct((NC, npad, d), jnp.float32),
        mesh=plsc.VectorSubcoreMesh(core_axis_name="c", subcore_axis_name="s",
                                    num_cores=NC, num_subcores=NS),
        scratch_types=[
            pltpu.VMEM((ns, ch), jnp.int32),
            pltpu.VMEM((ns, ch), jnp.int32),
        ] + [pltpu.VMEM((ch, d), jnp.float32)] * NBUF
          + [pltpu.SemaphoreType.DMA] * (2 * NBUF)
          + [pltpu.VMEM_SHARED((npad, d), jnp.float32)],
    )(g, row3, col3)


def _scale1_body(x_ref, w_ref, cnt_ref, g_ref):
    deg = cnt_ref[0] + cnt_ref[1] + 1.0
    dis = lax.rsqrt(deg)
    h = jnp.dot(x_ref[...], w_ref[...], preferred_element_type=jnp.float32)
    g_ref[...] = h * dis[:, None]


def _dense2_body(s_ref, g1_ref, cnt_ref, b_ref, w_ref, g2_ref):
    deg = cnt_ref[0] + cnt_ref[1] + 1.0
    dis = lax.rsqrt(deg)[:, None]
    t = (s_ref[0] + s_ref[1] + g1_ref[...]) * dis + b_ref[...]
    z = jnp.maximum(t, 0.0)
    g2_ref[...] = jnp.dot(z, w_ref[...],
                          preferred_element_type=jnp.float32) * dis


def _final_body(s_ref, g2_ref, cnt_ref, b_ref, out_ref):
    deg = cnt_ref[0] + cnt_ref[1] + 1.0
    dis = lax.rsqrt(deg)[:, None]
    out_ref[...] = (s_ref[0] + s_ref[1] + g2_ref[...]) * dis + b_ref[...]


def _scale1(x, W, cnt):
    npad, d = x.shape
    return pl.pallas_call(
        _scale1_body,
        grid=(npad // BN,),
        in_specs=[
            pl.BlockSpec((BN, d), lambda i: (i, 0)),
            pl.BlockSpec((d, d), lambda i: (0, 0)),
            pl.BlockSpec((NC, BN), lambda i: (0, i)),
        ],
        out_specs=pl.BlockSpec((BN, d), lambda i: (i, 0)),
        out_shape=jax.ShapeDtypeStruct((npad, d), jnp.float32),
    )(x, W, cnt)


def _dense2(s, g1, cnt, b, W):
    npad, d = g1.shape
    return pl.pallas_call(
        _dense2_body,
        grid=(npad // BN,),
        in_specs=[
            pl.BlockSpec((NC, BN, d), lambda i: (0, i, 0)),
            pl.BlockSpec((BN, d), lambda i: (i, 0)),
            pl.BlockSpec((NC, BN), lambda i: (0, i)),
            pl.BlockSpec((1, d), lambda i: (0, 0)),
            pl.BlockSpec((d, d), lambda i: (0, 0)),
        ],
        out_specs=pl.BlockSpec((BN, d), lambda i: (i, 0)),
        out_shape=jax.ShapeDtypeStruct((npad, d), jnp.float32),
    )(s, g1, cnt, b, W)


def _final(s, g2, cnt, b):
    npad, d = g2.shape
    return pl.pallas_call(
        _final_body,
        grid=(npad // BN,),
        in_specs=[
            pl.BlockSpec((NC, BN, d), lambda i: (0, i, 0)),
            pl.BlockSpec((BN, d), lambda i: (i, 0)),
            pl.BlockSpec((NC, BN), lambda i: (0, i)),
            pl.BlockSpec((1, d), lambda i: (0, 0)),
        ],
        out_specs=pl.BlockSpec((BN, d), lambda i: (i, 0)),
        out_shape=jax.ShapeDtypeStruct((npad, d), jnp.float32),
    )(s, g2, cnt, b)


def kernel(x, edge_index, W1, b1, W2, b2):
    n, d = x.shape
    e = edge_index.shape[1]
    epw = e // NW
    nsec = 5
    npad = -(-n // BN) * BN
    epp = -(-epw // (nsec * CH)) * (nsec * CH)
    ns = epp // CH // nsec
    row_w = edge_index[0].reshape(NW, epw)
    col_w = edge_index[1].reshape(NW, epw)
    row_w = jnp.pad(row_w, ((0, 0), (0, epp - epw)))
    col_w = jnp.pad(col_w, ((0, 0), (0, epp - epw)),
                    constant_values=npad - 1)
    row3 = row_w.reshape(NW, nsec, ns, CH)
    col3 = col_w.reshape(NW, nsec, ns, CH)
    xp = jnp.pad(x, ((0, npad - n), (0, 0)))

    cnt = _deg_call(col3, npad)
    g1 = _scale1(xp, W1, cnt)
    s1 = _scat_call(g1, row3, col3)
    g2 = _dense2(s1, g1, cnt, b1.reshape(1, d), W2)
    s2 = _scat_call(g2, row3, col3)
    return _final(s2, g2, cnt, b2.reshape(1, d))[:n]
